# baseline (speedup 1.0000x reference)
import jax
import jax.numpy as jnp
from jax import lax
from jax.experimental import pallas as pl
from jax.experimental.pallas import tpu as pltpu
from jax.experimental.pallas import tpu_sc as plsc

NC = 2
NS = 16
NW = NC * NS
PW = 256
GW = 256


def _pack_body(tableT, scratch, in0, in1, in_tail, out0, out1,
               si0, si1, so0, so1):
    wid = lax.axis_index("s") * NC + lax.axis_index("c")
    V = tableT.shape[1]
    nt = V // PW
    tail = V - nt * PW
    nunits = (nt + NW - 1) // NW
    iota = lax.iota(jnp.int32, 16)
    kvecs = [iota + 16 * g for g in range(4)]

    def start_in(j, inb, semb):
        t = jnp.minimum(wid + j * NW, nt - 1)
        pltpu.async_copy(tableT.at[:, pl.ds(t * PW, PW)], inb, semb)

    def wait_in(inb, semb):
        pltpu.make_async_copy(tableT.at[:, pl.ds(0, PW)], inb, semb).wait()

    def start_out(j, outb, semb):
        t = jnp.minimum(wid + j * NW, nt - 1)
        pltpu.async_copy(outb, scratch.at[pl.ds(t * (PW // 2), PW // 2), :], semb)

    def wait_out(outb, semb):
        pltpu.make_async_copy(outb, scratch.at[pl.ds(0, PW // 2), :], semb).wait()

    def transpose(src, dst, n_i):
        @plsc.parallel_loop(0, n_i // 4, unroll=2)
        def irow(i4):
            for a in range(4):
                i = i4 * 4 + a
                for g in range(4):
                    dst[i // 2, pl.ds((i % 2) * 64 + 16 * g, 16)] = (
                        plsc.load_gather(src, [kvecs[g], jnp.full((16,), i, jnp.int32)])
                    )

    start_in(0, in0, si0)
    start_in(1, in1, si1)
    wait_in(in0, si0)
    transpose(in0, out0, PW)
    start_out(0, out0, so0)
    start_in(2, in0, si0)
    wait_in(in1, si1)
    transpose(in1, out1, PW)
    start_out(1, out1, so1)
    start_in(3, in1, si1)

    def pair(p, carry):
        wait_in(in0, si0)
        wait_out(out0, so0)
        transpose(in0, out0, PW)
        start_out(2 * p, out0, so0)
        start_in(2 * p + 2, in0, si0)

        wait_in(in1, si1)
        wait_out(out1, so1)
        transpose(in1, out1, PW)
        start_out(2 * p + 1, out1, so1)

        @pl.when(p <= (nunits - 5) // 2)
        def _():
            start_in(2 * p + 3, in1, si1)
        return carry

    lax.fori_loop(1, (nunits - 1) // 2, pair, 0)

    wait_in(in0, si0)
    wait_out(out0, so0)
    transpose(in0, out0, PW)
    start_out(nunits - 1, out0, so0)
    wait_out(out0, so0)
    wait_out(out1, so1)

    if tail:
        @pl.when(wid == 0)
        def _():
            pltpu.async_copy(tableT.at[:, pl.ds(nt * PW, tail)], in_tail, si0).wait()
            transpose(in_tail, out0, tail)
            pltpu.async_copy(
                out0.at[pl.ds(0, tail // 2), :],
                scratch.at[pl.ds(nt * (PW // 2), tail // 2), :],
                so0,
            ).wait()


def _gather_body(scratchL, xT, out5,
                 idx0, idx1, rows0, rows1, blk0, blk1,
                 sx0, sx1, sr0, sr1, so0, so1):
    wid = lax.axis_index("s") * NC + lax.axis_index("c")
    H, B = xT.shape
    NBU = B // GW
    nunits = H * NBU // NW
    iota = lax.iota(jnp.int32, 16)
    bvecs = [iota + 16 * g + 128 * c for c in range(2) for g in range(8)]

    def start_idx(j, idxb, semb):
        u = wid + j * NW
        h = u // NBU
        bu = u - h * NBU
        pltpu.async_copy(xT.at[h, pl.ds(bu * GW, GW)], idxb, semb)

    def wait_idx(idxb, semb):
        pltpu.make_async_copy(xT.at[0, pl.ds(0, GW)], idxb, semb).wait()

    def start_gather(idxb, rowsb, semb):
        pltpu.async_copy(
            scratchL.at[idxb.at[pl.ds(0, 128)]], rowsb.at[pl.ds(0, 128), :], semb)
        pltpu.async_copy(
            scratchL.at[idxb.at[pl.ds(128, 128)]], rowsb.at[pl.ds(128, 128), :], semb)

    def wait_rows(rowsb, semb):
        pltpu.make_async_copy(scratchL.at[pl.ds(0, GW)], rowsb, semb).wait()

    def start_out(j, blkb, semb):
        u = wid + j * NW
        h = u // NBU
        bu = u - h * NBU
        pltpu.async_copy(blkb, out5.at[h, :, pl.ds(bu * 2, 2), :, :], semb)

    def wait_outw(blkb, semb):
        pltpu.make_async_copy(blkb, out5.at[0, :, pl.ds(0, 2), :, :], semb).wait()

    def transpose(rowsb, blkb):
        @plsc.parallel_loop(0, 64, unroll=2)
        def kl(k):
            kvec = jnp.full((16,), k, jnp.int32)
            for c in range(2):
                for g in range(8):
                    blkb[k // 8, c, k % 8, pl.ds(16 * g, 16)] = (
                        plsc.load_gather(rowsb, [bvecs[c * 8 + g], kvec])
                    )

    start_idx(0, idx0, sx0)
    start_idx(1, idx1, sx1)
    wait_idx(idx0, sx0)
    start_gather(idx0, rows0, sr0)
    wait_idx(idx1, sx1)
    start_gather(idx1, rows1, sr1)
    wait_rows(rows0, sr0)
    start_idx(2, idx0, sx0)
    transpose(rows0, blk0)
    start_out(0, blk0, so0)

    wait_idx(idx0, sx0)
    start_gather(idx0, rows0, sr0)
    wait_rows(rows1, sr1)
    start_idx(3, idx1, sx1)
    transpose(rows1, blk1)
    start_out(1, blk1, so1)

    def pair(p, carry):
        wait_idx(idx1, sx1)
        start_gather(idx1, rows1, sr1)
        wait_rows(rows0, sr0)
        wait_outw(blk0, so0)
        start_idx(2 * p + 2, idx0, sx0)
        transpose(rows0, blk0)
        start_out(2 * p, blk0, so0)

        wait_idx(idx0, sx0)
        start_gather(idx0, rows0, sr0)
        wait_rows(rows1, sr1)
        wait_outw(blk1, so1)
        start_idx(2 * p + 3, idx1, sx1)
        transpose(rows1, blk1)
        start_out(2 * p + 1, blk1, so1)
        return carry

    lax.fori_loop(1, nunits // 2 - 1, pair, 0)

    wait_idx(idx1, sx1)
    start_gather(idx1, rows1, sr1)
    wait_rows(rows0, sr0)
    wait_outw(blk0, so0)
    transpose(rows0, blk0)
    start_out(nunits - 2, blk0, so0)

    wait_rows(rows1, sr1)
    wait_outw(blk1, so1)
    transpose(rows1, blk1)
    start_out(nunits - 1, blk1, so1)

    wait_outw(blk0, so0)
    wait_outw(blk1, so1)


def kernel(x, table):
    V, D = table.shape
    B, H = x.shape
    NBT = B // 128
    assert D == 64 and V % (2 * PW) == 64 and B % GW == 0
    assert (H * B // GW) % (2 * NW) == 0
    tableT = table.T
    xT = x.T
    SR = V // 2 + 32

    mesh = plsc.VectorSubcoreMesh(core_axis_name="c", subcore_axis_name="s")
    params = pltpu.CompilerParams(
        use_tc_tiling_on_sc=True, needs_layout_passes=False
    )
    params_lin = pltpu.CompilerParams(
        use_tc_tiling_on_sc=False, needs_layout_passes=False
    )

    pack = pl.kernel(
        _pack_body,
        out_type=jax.ShapeDtypeStruct((SR, 128), jnp.float32),
        mesh=mesh,
        scratch_types=[
            pltpu.VMEM((64, PW), jnp.float32),
            pltpu.VMEM((64, PW), jnp.float32),
            pltpu.VMEM((64, 64), jnp.float32),
            pltpu.VMEM((PW // 2, 128), jnp.float32),
            pltpu.VMEM((PW // 2, 128), jnp.float32),
            pltpu.SemaphoreType.DMA,
            pltpu.SemaphoreType.DMA,
            pltpu.SemaphoreType.DMA,
            pltpu.SemaphoreType.DMA,
        ],
        compiler_params=params,
    )
    scratch = pack(tableT)
    scratchL = scratch.reshape(2 * SR, D)

    gat = pl.kernel(
        _gather_body,
        out_type=jax.ShapeDtypeStruct((H, 8, NBT, 8, 128), jnp.float32),
        mesh=mesh,
        scratch_types=[
            pltpu.VMEM((GW,), jnp.int32),
            pltpu.VMEM((GW,), jnp.int32),
            pltpu.VMEM((GW, 64), jnp.float32),
            pltpu.VMEM((GW, 64), jnp.float32),
            pltpu.VMEM((8, 2, 8, 128), jnp.float32),
            pltpu.VMEM((8, 2, 8, 128), jnp.float32),
            pltpu.SemaphoreType.DMA,
            pltpu.SemaphoreType.DMA,
            pltpu.SemaphoreType.DMA,
            pltpu.SemaphoreType.DMA,
            pltpu.SemaphoreType.DMA,
            pltpu.SemaphoreType.DMA,
        ],
        compiler_params=params_lin,
    )
    out5 = gat(scratchL, xT)
    return out5.transpose(2, 4, 0, 1, 3).reshape(B, H, D)


# speedup vs baseline: 1.1631x; 1.1631x over previous
import functools

import jax
import jax.numpy as jnp
from jax import lax
from jax.experimental import pallas as pl
from jax.experimental.pallas import tpu as pltpu
from jax.experimental.pallas import tpu_sc as plsc

D = 64
NC = 2
NS = 16
NW = NC * NS
G = 128
K = 4
CHUNK = K * G


def _emb_body(b_per_w, table, idx_hbm, out,
              idx0, idx1, rows0, rows1, si0, si1, sg0, sg1, ss0, ss1):
    wid = lax.axis_index("s") * NC + lax.axis_index("c")
    base = wid * b_per_w
    nblk = b_per_w // CHUNK

    def idx_load(i, idxb, semb):
        pltpu.async_copy(idx_hbm.at[pl.ds(base + i * CHUNK, CHUNK)], idxb, semb)

    def wait_idx(idxb, semb):
        pltpu.make_async_copy(idx_hbm.at[pl.ds(0, CHUNK)], idxb, semb).wait()

    def fire(idxb, rowsb, semb):
        for j in range(K):
            pltpu.async_copy(
                table.at[idxb.at[pl.ds(j * G, G)]],
                rowsb.at[pl.ds(j * G, G)],
                semb,
            )

    def wait_g(rowsb, semb):
        pltpu.make_async_copy(table.at[pl.ds(0, CHUNK)], rowsb, semb).wait()

    def store(i, rowsb, semb):
        pltpu.async_copy(rowsb, out.at[pl.ds(base + i * CHUNK, CHUNK)], semb)

    def wait_store(rowsb, semb):
        pltpu.make_async_copy(rowsb, out.at[pl.ds(0, CHUNK)], semb).wait()

    idx_load(0, idx0, si0)
    idx_load(1, idx1, si1)
    wait_idx(idx0, si0)
    fire(idx0, rows0, sg0)
    wait_idx(idx1, si1)
    fire(idx1, rows1, sg1)
    wait_g(rows0, sg0)
    store(0, rows0, ss0)
    idx_load(2, idx0, si0)

    def pair(p, carry):
        i0 = 2 * p
        wait_idx(idx0, si0)
        wait_store(rows0, ss0)
        fire(idx0, rows0, sg0)
        wait_g(rows1, sg1)
        store(i0 - 1, rows1, ss1)
        idx_load(i0 + 1, idx1, si1)

        wait_idx(idx1, si1)
        wait_store(rows1, ss1)
        fire(idx1, rows1, sg1)
        wait_g(rows0, sg0)
        store(i0, rows0, ss0)
        idx_load(i0 + 2, idx0, si0)
        return carry

    lax.fori_loop(1, nblk // 2 - 1, pair, 0)

    i0 = nblk - 2
    wait_idx(idx0, si0)
    wait_store(rows0, ss0)
    fire(idx0, rows0, sg0)
    wait_g(rows1, sg1)
    store(i0 - 1, rows1, ss1)
    idx_load(i0 + 1, idx1, si1)

    wait_idx(idx1, si1)
    wait_store(rows1, ss1)
    fire(idx1, rows1, sg1)
    wait_g(rows0, sg0)
    store(i0, rows0, ss0)

    wait_g(rows1, sg1)
    store(i0 + 1, rows1, ss1)
    wait_store(rows0, ss0)
    wait_store(rows1, ss1)


def kernel(x, table):
    B, H = x.shape
    total = B * H
    b_per_w = total // NW
    assert total % NW == 0
    nblk = b_per_w // CHUNK
    assert b_per_w % CHUNK == 0 and nblk % 2 == 0 and nblk >= 4
    idx_flat = x.reshape(total)

    mesh = plsc.VectorSubcoreMesh(core_axis_name="c", subcore_axis_name="s")
    emb = pl.kernel(
        functools.partial(_emb_body, b_per_w),
        out_type=jax.ShapeDtypeStruct((total, D), jnp.float32),
        mesh=mesh,
        scratch_types=[
            pltpu.VMEM((CHUNK,), jnp.int32),
            pltpu.VMEM((CHUNK,), jnp.int32),
            pltpu.VMEM((CHUNK, D), jnp.float32),
            pltpu.VMEM((CHUNK, D), jnp.float32),
            pltpu.SemaphoreType.DMA,
            pltpu.SemaphoreType.DMA,
            pltpu.SemaphoreType.DMA,
            pltpu.SemaphoreType.DMA,
            pltpu.SemaphoreType.DMA,
            pltpu.SemaphoreType.DMA,
        ],
        compiler_params=pltpu.CompilerParams(use_tc_tiling_on_sc=False),
    )
    out_flat = emb(table, idx_flat)
    return out_flat.reshape(B, H, D)
